# Initial kernel scaffold; baseline (speedup 1.0000x reference)
import jax
import jax.numpy as jnp
from jax import lax
from jax.experimental import pallas as pl
from jax.experimental.pallas import tpu as pltpu
from jax.experimental.pallas import tpu_sc as plsc

N_NODES = 10000
N_PAD = 10240
N_EDGES = 320000
D = 128
NC, NS, L = 2, 16, 16
NW = NC * NS
E_PER_W = N_EDGES // NW
CHUNK = 80
NCHUNK = E_PER_W // CHUNK
ROWS_PER_TILE = N_PAD // NS
DEG_W = 16
TC_BLK = 2048


def _sc_body(x_hbm, ei_hbm, agg_hbm, deg_hbm,
             agg_sh, deg_sh, idx_v, rows_v, ones_v, sem):
    c = lax.axis_index("c")
    s = lax.axis_index("s")
    wid = s * NC + c
    base_r = s * ROWS_PER_TILE

    zero16 = jnp.zeros((L,), jnp.float32)

    def _zrows(i, carry):
        for k in range(D // L):
            rows_v[i, pl.ds(k * L, L)] = zero16
        return carry

    lax.fori_loop(0, CHUNK, _zrows, 0)
    for k in range(ROWS_PER_TILE // CHUNK):
        pltpu.sync_copy(rows_v, agg_sh.at[pl.ds(base_r + k * CHUNK, CHUNK)])

    def _zones(i, carry):
        ones_v[i, :] = zero16
        return carry

    lax.fori_loop(0, CHUNK, _zones, 0)
    for k in range(ROWS_PER_TILE // CHUNK):
        pltpu.sync_copy(ones_v, deg_sh.at[pl.ds(base_r + k * CHUNK, CHUNK)])

    one16 = jnp.ones((L,), jnp.float32)

    def _ones(i, carry):
        ones_v[i, :] = one16
        return carry

    lax.fori_loop(0, CHUNK, _ones, 0)
    plsc.subcore_barrier()

    ebase = wid * E_PER_W

    def _chunk(ci, carry):
        off = ebase + ci * CHUNK
        pltpu.sync_copy(ei_hbm.at[0, pl.ds(off, CHUNK)], idx_v.at[0])
        pltpu.sync_copy(ei_hbm.at[1, pl.ds(off, CHUNK)], idx_v.at[1])
        pltpu.async_copy(x_hbm.at[idx_v.at[0]], rows_v, sem).wait()
        pltpu.sync_copy(rows_v, agg_sh.at[idx_v.at[1]], add=True)
        pltpu.sync_copy(ones_v, deg_sh.at[idx_v.at[1]], add=True)
        return carry

    lax.fori_loop(0, NCHUNK, _chunk, 0)
    plsc.subcore_barrier()

    pltpu.sync_copy(agg_sh.at[pl.ds(base_r, ROWS_PER_TILE)],
                    agg_hbm.at[c, pl.ds(base_r, ROWS_PER_TILE)])
    pltpu.sync_copy(deg_sh.at[pl.ds(base_r, ROWS_PER_TILE)],
                    deg_hbm.at[c, pl.ds(base_r, ROWS_PER_TILE)])


_sc_aggregate = pl.kernel(
    _sc_body,
    out_type=(
        jax.ShapeDtypeStruct((NC, N_PAD, D), jnp.float32),
        jax.ShapeDtypeStruct((NC, N_PAD, DEG_W), jnp.float32),
    ),
    mesh=plsc.VectorSubcoreMesh(core_axis_name="c", subcore_axis_name="s"),
    scratch_types=(
        pltpu.VMEM_SHARED((N_PAD, D), jnp.float32),
        pltpu.VMEM_SHARED((N_PAD, DEG_W), jnp.float32),
        pltpu.VMEM((2, CHUNK), jnp.int32),
        pltpu.VMEM((CHUNK, D), jnp.float32),
        pltpu.VMEM((CHUNK, DEG_W), jnp.float32),
        pltpu.SemaphoreType.DMA,
    ),
)


def _tc_body(agg_ref, deg_ref, w_ref, b_ref, out_ref):
    a = agg_ref[0] + agg_ref[1]
    dg = deg_ref[0] + deg_ref[1]
    dg = jnp.maximum(dg[:, 0:1], 1.0)
    a = a / dg
    out_ref[...] = (
        jnp.dot(a, w_ref[...], preferred_element_type=jnp.float32) + b_ref[...]
    )


def _tc_finish(agg, deg, W, b2):
    return pl.pallas_call(
        _tc_body,
        grid=(N_PAD // TC_BLK,),
        in_specs=[
            pl.BlockSpec((NC, TC_BLK, D), lambda i: (0, i, 0)),
            pl.BlockSpec((NC, TC_BLK, DEG_W), lambda i: (0, i, 0)),
            pl.BlockSpec((D, D), lambda i: (0, 0)),
            pl.BlockSpec((1, D), lambda i: (0, 0)),
        ],
        out_specs=pl.BlockSpec((TC_BLK, D), lambda i: (i, 0)),
        out_shape=jax.ShapeDtypeStruct((N_PAD, D), jnp.float32),
    )(agg, deg, W, b2)


def kernel(x, edge_index, W, b):
    agg, deg = _sc_aggregate(x, edge_index)
    out = _tc_finish(agg, deg, W, b.reshape(1, D))
    return out[:N_NODES]


# speedup vs baseline: 6.0488x; 6.0488x over previous
import jax
import jax.numpy as jnp
from jax import lax
from jax.experimental import pallas as pl
from jax.experimental.pallas import tpu as pltpu
from jax.experimental.pallas import tpu_sc as plsc

N_NODES = 10000
N_PAD = 10240
N_EDGES = 320000
D = 128
NC, NS, L = 2, 16, 16
NW = NC * NS
E_PER_W = N_EDGES // NW
CHUNK = 80
NCHUNK = E_PER_W // CHUNK
ROWS_PER_TILE = N_PAD // NS
TC_BLK = 2048


def _sc_body(x_hbm, src_hbm, dst_hbm, z128_hbm, zo1_hbm,
             agg_hbm, deg_hbm,
             agg_sh, deg_sh, sidx_v, didx_v, rows_v, ones_v, sem):
    c = lax.axis_index("c")
    s = lax.axis_index("s")
    wid = s * NC + c
    base_r = s * ROWS_PER_TILE

    pltpu.sync_copy(z128_hbm, rows_v)
    for k in range(ROWS_PER_TILE // CHUNK):
        pltpu.sync_copy(rows_v, agg_sh.at[pl.ds(base_r + k * CHUNK, CHUNK)])

    pltpu.sync_copy(zo1_hbm.at[pl.ds(0, ROWS_PER_TILE)],
                    deg_sh.at[pl.ds(base_r, ROWS_PER_TILE)])
    pltpu.sync_copy(zo1_hbm.at[pl.ds(ROWS_PER_TILE, CHUNK)], ones_v)
    plsc.subcore_barrier()

    ebase = wid * E_PER_W

    def _chunk(ci, carry):
        off = ebase + ci * CHUNK
        pltpu.sync_copy(src_hbm.at[pl.ds(off, CHUNK)], sidx_v)
        pltpu.sync_copy(dst_hbm.at[pl.ds(off, CHUNK)], didx_v)
        pltpu.async_copy(x_hbm.at[sidx_v], rows_v, sem).wait()
        pltpu.sync_copy(rows_v, agg_sh.at[didx_v], add=True)
        pltpu.sync_copy(ones_v, deg_sh.at[didx_v], add=True)
        return carry

    lax.fori_loop(0, NCHUNK, _chunk, 0)
    plsc.subcore_barrier()

    out_r = c * N_PAD + base_r
    pltpu.sync_copy(agg_sh.at[pl.ds(base_r, ROWS_PER_TILE)],
                    agg_hbm.at[pl.ds(out_r, ROWS_PER_TILE)])
    pltpu.sync_copy(deg_sh.at[pl.ds(base_r, ROWS_PER_TILE)],
                    deg_hbm.at[pl.ds(out_r, ROWS_PER_TILE)])


_sc_aggregate = pl.kernel(
    _sc_body,
    out_type=(
        jax.ShapeDtypeStruct((NC * N_PAD, D), jnp.float32),
        jax.ShapeDtypeStruct((NC * N_PAD,), jnp.float32),
    ),
    mesh=plsc.VectorSubcoreMesh(core_axis_name="c", subcore_axis_name="s"),
    scratch_types=(
        pltpu.VMEM_SHARED((N_PAD, D), jnp.float32),
        pltpu.VMEM_SHARED((N_PAD,), jnp.float32),
        pltpu.VMEM((CHUNK,), jnp.int32),
        pltpu.VMEM((CHUNK,), jnp.int32),
        pltpu.VMEM((CHUNK, D), jnp.float32),
        pltpu.VMEM((CHUNK,), jnp.float32),
        pltpu.SemaphoreType.DMA,
    ),
)


def _tc_body(agg_ref, deg_ref, w_ref, b_ref, out_ref):
    a = agg_ref[0] + agg_ref[1]
    dg = deg_ref[0] + deg_ref[1]
    dg = jnp.maximum(dg, 1.0)
    a = a / dg
    out_ref[...] = (
        jnp.dot(a, w_ref[...], preferred_element_type=jnp.float32) + b_ref[...]
    )


def _tc_finish(agg, deg, W, b2):
    return pl.pallas_call(
        _tc_body,
        grid=(N_PAD // TC_BLK,),
        in_specs=[
            pl.BlockSpec((NC, TC_BLK, D), lambda i: (0, i, 0)),
            pl.BlockSpec((NC, TC_BLK, 1), lambda i: (0, i, 0)),
            pl.BlockSpec((D, D), lambda i: (0, 0)),
            pl.BlockSpec((1, D), lambda i: (0, 0)),
        ],
        out_specs=pl.BlockSpec((TC_BLK, D), lambda i: (i, 0)),
        out_shape=jax.ShapeDtypeStruct((N_PAD, D), jnp.float32),
    )(agg, deg, W, b2)


def kernel(x, edge_index, W, b):
    z128 = jnp.zeros((CHUNK, D), jnp.float32)
    zo1 = jnp.concatenate(
        [jnp.zeros((ROWS_PER_TILE,), jnp.float32),
         jnp.ones((CHUNK,), jnp.float32)], axis=0)
    agg, deg = _sc_aggregate(x, edge_index[0], edge_index[1], z128, zo1)
    agg = agg.reshape(NC, N_PAD, D)
    deg = deg.reshape(NC, N_PAD, 1)
    out = _tc_finish(agg, deg, W, b.reshape(1, D))
    return out[:N_NODES]


# speedup vs baseline: 12.3738x; 2.0457x over previous
import jax
import jax.numpy as jnp
from jax import lax
from jax.experimental import pallas as pl
from jax.experimental.pallas import tpu as pltpu
from jax.experimental.pallas import tpu_sc as plsc

N_NODES = 10000
N_PAD = 10240
N_EDGES = 320000
D = 128
NC, NS, L = 2, 16, 16
NW = NC * NS
E_PER_W = N_EDGES // NW
CHUNK = 80
NCHUNK = E_PER_W // CHUNK
NPAIR = (NCHUNK - 1) // 2
ROWS_PER_TILE = N_PAD // NS
TC_BLK = 2048


def _sc_body(x_hbm, src_hbm, dst_hbm, z128_hbm, zo1_hbm,
             agg_hbm, deg_hbm,
             agg_sh, deg_sh, sidx_v, didx_v, rows0_v, rows1_v, ones_v,
             sem0, sem1):
    c = lax.axis_index("c")
    s = lax.axis_index("s")
    wid = s * NC + c
    base_r = s * ROWS_PER_TILE

    pltpu.sync_copy(z128_hbm, rows0_v)
    for k in range(ROWS_PER_TILE // CHUNK):
        pltpu.sync_copy(rows0_v, agg_sh.at[pl.ds(base_r + k * CHUNK, CHUNK)])

    pltpu.sync_copy(zo1_hbm.at[pl.ds(0, ROWS_PER_TILE)],
                    deg_sh.at[pl.ds(base_r, ROWS_PER_TILE)])
    pltpu.sync_copy(zo1_hbm.at[pl.ds(ROWS_PER_TILE, CHUNK)], ones_v)

    pltpu.sync_copy(src_hbm.at[wid], sidx_v)
    pltpu.sync_copy(dst_hbm.at[wid], didx_v)
    plsc.subcore_barrier()

    def _gather(ci, rows, sem):
        return pltpu.async_copy(
            x_hbm.at[sidx_v.at[pl.ds(ci * CHUNK, CHUNK)]], rows, sem)

    def _scatter(ci, rows):
        pltpu.sync_copy(rows, agg_sh.at[didx_v.at[ci]], add=True)
        pltpu.sync_copy(ones_v, deg_sh.at[didx_v.at[ci]], add=True)

    _gather(0, rows0_v, sem0)

    def _pair(g, carry):
        c0 = 2 * g
        _gather(c0 + 1, rows1_v, sem1)
        pltpu.make_async_copy(x_hbm.at[pl.ds(0, CHUNK)], rows0_v, sem0).wait()
        _scatter(c0, rows0_v)
        _gather(c0 + 2, rows0_v, sem0)
        pltpu.make_async_copy(x_hbm.at[pl.ds(0, CHUNK)], rows1_v, sem1).wait()
        _scatter(c0 + 1, rows1_v)
        return carry

    lax.fori_loop(0, NPAIR, _pair, 0)
    pltpu.make_async_copy(x_hbm.at[pl.ds(0, CHUNK)], rows0_v, sem0).wait()
    _scatter(NCHUNK - 1, rows0_v)
    plsc.subcore_barrier()

    out_r = c * N_PAD + base_r
    pltpu.sync_copy(agg_sh.at[pl.ds(base_r, ROWS_PER_TILE)],
                    agg_hbm.at[pl.ds(out_r, ROWS_PER_TILE)])
    pltpu.sync_copy(deg_sh.at[pl.ds(base_r, ROWS_PER_TILE)],
                    deg_hbm.at[pl.ds(out_r, ROWS_PER_TILE)])


_sc_aggregate = pl.kernel(
    _sc_body,
    out_type=(
        jax.ShapeDtypeStruct((NC * N_PAD, D), jnp.float32),
        jax.ShapeDtypeStruct((NC * N_PAD,), jnp.float32),
    ),
    mesh=plsc.VectorSubcoreMesh(core_axis_name="c", subcore_axis_name="s"),
    scratch_types=(
        pltpu.VMEM_SHARED((N_PAD, D), jnp.float32),
        pltpu.VMEM_SHARED((N_PAD,), jnp.float32),
        pltpu.VMEM((E_PER_W,), jnp.int32),
        pltpu.VMEM((NCHUNK, CHUNK), jnp.int32),
        pltpu.VMEM((CHUNK, D), jnp.float32),
        pltpu.VMEM((CHUNK, D), jnp.float32),
        pltpu.VMEM((CHUNK,), jnp.float32),
        pltpu.SemaphoreType.DMA,
        pltpu.SemaphoreType.DMA,
    ),
)


def _tc_body(agg_ref, deg_ref, w_ref, b_ref, out_ref):
    a = agg_ref[0] + agg_ref[1]
    dg = deg_ref[0] + deg_ref[1]
    dg = jnp.maximum(dg, 1.0)
    a = a / dg
    out_ref[...] = (
        jnp.dot(a, w_ref[...], preferred_element_type=jnp.float32) + b_ref[...]
    )


def _tc_finish(agg, deg, W, b2):
    return pl.pallas_call(
        _tc_body,
        grid=(N_PAD // TC_BLK,),
        in_specs=[
            pl.BlockSpec((NC, TC_BLK, D), lambda i: (0, i, 0)),
            pl.BlockSpec((NC, TC_BLK, 1), lambda i: (0, i, 0)),
            pl.BlockSpec((D, D), lambda i: (0, 0)),
            pl.BlockSpec((1, D), lambda i: (0, 0)),
        ],
        out_specs=pl.BlockSpec((TC_BLK, D), lambda i: (i, 0)),
        out_shape=jax.ShapeDtypeStruct((N_PAD, D), jnp.float32),
    )(agg, deg, W, b2)


def kernel(x, edge_index, W, b):
    z128 = jnp.zeros((CHUNK, D), jnp.float32)
    zo1 = jnp.concatenate(
        [jnp.zeros((ROWS_PER_TILE,), jnp.float32),
         jnp.ones((CHUNK,), jnp.float32)], axis=0)
    src = edge_index[0].reshape(NW, E_PER_W)
    dst = edge_index[1].reshape(NW, NCHUNK, CHUNK)
    agg, deg = _sc_aggregate(x, src, dst, z128, zo1)
    agg = agg.reshape(NC, N_PAD, D)
    deg = deg.reshape(NC, N_PAD, 1)
    out = _tc_finish(agg, deg, W, b.reshape(1, D))
    return out[:N_NODES]


# speedup vs baseline: 13.0826x; 1.0573x over previous
import jax
import jax.numpy as jnp
from jax import lax
from jax.experimental import pallas as pl
from jax.experimental.pallas import tpu as pltpu
from jax.experimental.pallas import tpu_sc as plsc

N_NODES = 10000
N_PAD = 10240
N_EDGES = 320000
D = 128
NC, NS, L = 2, 16, 16
NW = NC * NS
CHUNK = 128
E_PER_W = 10240
E_TOTAL = NW * E_PER_W
NHALF = 2
E_PER_H = E_PER_W // NHALF
NCHUNK_H = E_PER_H // CHUNK
NPAIR_H = NCHUNK_H // 2 - 1
ROWS_PER_TILE = N_PAD // NS
ZCHUNK = 80
TC_BLK = 2048


def _sc_body(x_hbm, src_hbm, dst_hbm, z128_hbm, zo1_hbm,
             agg_hbm, deg_hbm,
             agg_sh, deg_sh, sidx_v, didx_v, rows0_v, rows1_v, ones_v,
             sem0, sem1):
    c = lax.axis_index("c")
    s = lax.axis_index("s")
    wid = s * NC + c
    base_r = s * ROWS_PER_TILE

    pltpu.sync_copy(z128_hbm, rows0_v)
    for k in range(ROWS_PER_TILE // ZCHUNK):
        pltpu.sync_copy(rows0_v.at[pl.ds(0, ZCHUNK)],
                        agg_sh.at[pl.ds(base_r + k * ZCHUNK, ZCHUNK)])

    pltpu.sync_copy(zo1_hbm.at[pl.ds(0, ROWS_PER_TILE)],
                    deg_sh.at[pl.ds(base_r, ROWS_PER_TILE)])
    pltpu.sync_copy(zo1_hbm.at[pl.ds(ROWS_PER_TILE, CHUNK)], ones_v)

    plsc.subcore_barrier()

    def _gather(ci, rows, sem):
        return pltpu.async_copy(
            x_hbm.at[sidx_v.at[pl.ds(ci * CHUNK, CHUNK)]], rows, sem)

    def _wait(rows, sem):
        pltpu.make_async_copy(x_hbm.at[pl.ds(0, CHUNK)], rows, sem).wait()

    def _scatter(ci, rows):
        pltpu.sync_copy(rows, agg_sh.at[didx_v.at[ci]], add=True)
        pltpu.sync_copy(ones_v, deg_sh.at[didx_v.at[ci]], add=True)

    def _pair(g, carry):
        c0 = 2 * g
        _gather(c0 + 1, rows1_v, sem1)
        _wait(rows0_v, sem0)
        _scatter(c0, rows0_v)
        _gather(c0 + 2, rows0_v, sem0)
        _wait(rows1_v, sem1)
        _scatter(c0 + 1, rows1_v)
        return carry

    for h in range(NHALF):
        hid = wid * NHALF + h
        pltpu.sync_copy(src_hbm.at[hid], sidx_v)
        pltpu.sync_copy(dst_hbm.at[hid], didx_v)
        _gather(0, rows0_v, sem0)
        lax.fori_loop(0, NPAIR_H, _pair, 0)
        _gather(NCHUNK_H - 1, rows1_v, sem1)
        _wait(rows0_v, sem0)
        _scatter(NCHUNK_H - 2, rows0_v)
        _wait(rows1_v, sem1)
        _scatter(NCHUNK_H - 1, rows1_v)
    plsc.subcore_barrier()

    out_r = c * N_PAD + base_r
    pltpu.sync_copy(agg_sh.at[pl.ds(base_r, ROWS_PER_TILE)],
                    agg_hbm.at[pl.ds(out_r, ROWS_PER_TILE)])
    pltpu.sync_copy(deg_sh.at[pl.ds(base_r, ROWS_PER_TILE)],
                    deg_hbm.at[pl.ds(out_r, ROWS_PER_TILE)])


_sc_aggregate = pl.kernel(
    _sc_body,
    out_type=(
        jax.ShapeDtypeStruct((NC * N_PAD, D), jnp.float32),
        jax.ShapeDtypeStruct((NC * N_PAD,), jnp.float32),
    ),
    mesh=plsc.VectorSubcoreMesh(core_axis_name="c", subcore_axis_name="s"),
    scratch_types=(
        pltpu.VMEM_SHARED((N_PAD, D), jnp.float32),
        pltpu.VMEM_SHARED((N_PAD,), jnp.float32),
        pltpu.VMEM((E_PER_H,), jnp.int32),
        pltpu.VMEM((NCHUNK_H, CHUNK), jnp.int32),
        pltpu.VMEM((CHUNK, D), jnp.float32),
        pltpu.VMEM((CHUNK, D), jnp.float32),
        pltpu.VMEM((CHUNK,), jnp.float32),
        pltpu.SemaphoreType.DMA,
        pltpu.SemaphoreType.DMA,
    ),
)


def _tc_body(agg_ref, deg_ref, w_ref, b_ref, out_ref):
    a = agg_ref[0] + agg_ref[1]
    dg = deg_ref[0] + deg_ref[1]
    dg = jnp.maximum(dg, 1.0)
    a = a / dg
    out_ref[...] = (
        jnp.dot(a, w_ref[...], preferred_element_type=jnp.float32) + b_ref[...]
    )


def _tc_finish(agg, deg, W, b2):
    return pl.pallas_call(
        _tc_body,
        grid=(N_PAD // TC_BLK,),
        in_specs=[
            pl.BlockSpec((NC, TC_BLK, D), lambda i: (0, i, 0)),
            pl.BlockSpec((NC, TC_BLK, 1), lambda i: (0, i, 0)),
            pl.BlockSpec((D, D), lambda i: (0, 0)),
            pl.BlockSpec((1, D), lambda i: (0, 0)),
        ],
        out_specs=pl.BlockSpec((TC_BLK, D), lambda i: (i, 0)),
        out_shape=jax.ShapeDtypeStruct((N_PAD, D), jnp.float32),
    )(agg, deg, W, b2)


def kernel(x, edge_index, W, b):
    z128 = jnp.zeros((CHUNK, D), jnp.float32)
    zo1 = jnp.concatenate(
        [jnp.zeros((ROWS_PER_TILE,), jnp.float32),
         jnp.ones((CHUNK,), jnp.float32)], axis=0)
    npad = E_TOTAL - N_EDGES
    pad_i = jnp.arange(npad, dtype=jnp.int32)
    src = jnp.concatenate([edge_index[0], pad_i % N_NODES])
    dst = jnp.concatenate([edge_index[1], N_NODES + pad_i % (N_PAD - N_NODES)])
    agg, deg = _sc_aggregate(x, src.reshape(NW * NHALF, E_PER_H),
                             dst.reshape(NW * NHALF, NCHUNK_H, CHUNK),
                             z128, zo1)
    agg = agg.reshape(NC, N_PAD, D)
    deg = deg.reshape(NC, N_PAD, 1)
    out = _tc_finish(agg, deg, W, b.reshape(1, D))
    return out[:N_NODES]
